# baseline (speedup 1.0000x reference)
import jax
import jax.numpy as jnp
from jax import lax
from jax.experimental import pallas as pl
from jax.experimental.pallas import tpu as pltpu
from jax.experimental.pallas import tpu_sc as plsc

B, S, D = 4, 4096, 1024
SC_B = 1
TC_B = B - SC_B
NC, NS = 2, 16
NW = NC * NS
SW = S // NW
R = 16
NSC = SW // R
STEPS = NSC * SC_B
NBUF = 5
PF = 3


def _sc_body(x_hbm, pos_hbm, out_hbm,
             bx0, bx1, bx2, bx3, bx4, bp0, bp1,
             si0, si1, si2, si3, si4, so0, so1, so2, so3, so4, sp0, sp1):
    bx = (bx0, bx1, bx2, bx3, bx4)
    bp = (bp0, bp1)
    si = (si0, si1, si2, si3, si4)
    so = (so0, so1, so2, so3, so4)
    sp = (sp0, sp1)
    wid = lax.axis_index("s") * NC + lax.axis_index("c")
    s0 = wid * SW

    steps = [(sc_i, b) for sc_i in range(NSC) for b in range(SC_B)]

    def x_rows(k):
        sc_i, b = steps[k]
        return pl.ds((TC_B + b) * S + s0 + sc_i * R, R)

    def out_rows(k):
        sc_i, b = steps[k]
        return pl.ds(b * S + s0 + sc_i * R, R)

    def pos_rows(sc_i):
        return pl.ds(s0 + sc_i * R, R)

    pltpu.async_copy(pos_hbm.at[pos_rows(steps[0][0])], bp[0], sp[0])
    if STEPS > 1:
        pltpu.async_copy(pos_hbm.at[pos_rows(steps[1][0])], bp[1], sp[1])
    for k in range(PF):
        pltpu.async_copy(x_hbm.at[x_rows(k)], bx[k % NBUF], si[k % NBUF])

    for k in range(STEPS):
        sc_i, b = steps[k]
        pltpu.make_async_copy(
            x_hbm.at[x_rows(k)], bx[k % NBUF], si[k % NBUF]).wait()
        pltpu.make_async_copy(
            pos_hbm.at[pos_rows(sc_i)], bp[k % 2], sp[k % 2]).wait()

        xb = bx[k % NBUF]
        pb = bp[k % 2]

        @plsc.parallel_loop(0, R * D, step=16, unroll=8)
        def _add(i):
            r = i >> 10
            c = pl.multiple_of(i & (D - 1), 16)
            plsc.addupdate(xb.at[r, pl.ds(c, 16)], pb[r, pl.ds(c, 16)])

        pltpu.async_copy(xb, out_hbm.at[out_rows(k)], so[k % NBUF])

        if k + 2 < STEPS:
            pltpu.async_copy(
                pos_hbm.at[pos_rows(steps[k + 2][0])], bp[k % 2], sp[k % 2])

        nk = k + PF
        if nk < STEPS:
            ko = nk - NBUF
            if ko >= 0:
                pltpu.make_async_copy(
                    bx[ko % NBUF], out_hbm.at[out_rows(ko)],
                    so[ko % NBUF]).wait()
            pltpu.async_copy(x_hbm.at[x_rows(nk)], bx[nk % NBUF], si[nk % NBUF])

    for k in range(max(0, STEPS - NBUF), STEPS):
        pltpu.make_async_copy(
            bx[k % NBUF], out_hbm.at[out_rows(k)], so[k % NBUF]).wait()


def _tc_add_body(x_ref, pos_ref, o_ref):
    o_ref[...] = x_ref[...] + pos_ref[...]


SB = 2048


@jax.jit
def _hybrid(x2, pos_table):
    o_tc = pl.pallas_call(
        _tc_add_body,
        grid=(S // SB, TC_B),
        in_specs=[
            pl.BlockSpec((1, SB, D), lambda s, b: (b, s, 0)),
            pl.BlockSpec((SB, D), lambda s, b: (s, 0)),
        ],
        out_specs=pl.BlockSpec((1, SB, D), lambda s, b: (b, s, 0)),
        out_shape=jax.ShapeDtypeStruct((TC_B, S, D), jnp.float32),
    )(x2.reshape(B, S, D), pos_table)

    mesh = plsc.VectorSubcoreMesh(core_axis_name="c", subcore_axis_name="s")
    o_sc = pl.kernel(
        _sc_body,
        out_type=jax.ShapeDtypeStruct((SC_B * S, D), jnp.float32),
        mesh=mesh,
        scratch_types=(
            [pltpu.VMEM((R, D), jnp.float32)] * (NBUF + 2)
            + [pltpu.SemaphoreType.DMA] * (2 * NBUF + 2)
        ),
        compiler_params=pltpu.CompilerParams(use_tc_tiling_on_sc=True),
    )(x2, pos_table)

    return jnp.concatenate([o_tc, o_sc.reshape(SC_B, S, D)], axis=0)


def kernel(x, pos_table):
    return _hybrid(x.reshape(B * S, D), pos_table)


# speedup vs baseline: 1.3601x; 1.3601x over previous
import jax
import jax.numpy as jnp
from jax import lax
from jax.experimental import pallas as pl
from jax.experimental.pallas import tpu as pltpu
from jax.experimental.pallas import tpu_sc as plsc

B, S, D = 4, 4096, 1024
NC, NS = 2, 16
NW = NC * NS
SW = S // NW
R = 32
NSC = SW // R
STEPS = NSC * B
NBUF = 3
PF = 2


def _sc_body(x_hbm, pos_hbm, out_hbm,
             bx0, bx1, bx2, bp0,
             si0, si1, si2, so0, so1, so2, sp0):
    bx = (bx0, bx1, bx2)
    si = (si0, si1, si2)
    so = (so0, so1, so2)
    wid = lax.axis_index("s") * NC + lax.axis_index("c")
    s0 = wid * SW

    steps = [(sc_i, b) for sc_i in range(NSC) for b in range(B)]

    def x_rows(k):
        sc_i, b = steps[k]
        return pl.ds(b * S + s0 + sc_i * R, R)

    def pos_rows(sc_i):
        return pl.ds(s0 + sc_i * R, R)

    pltpu.async_copy(pos_hbm.at[pos_rows(0)], bp0, sp0)
    for k in range(PF):
        pltpu.async_copy(x_hbm.at[x_rows(k)], bx[k % NBUF], si[k % NBUF])

    for k in range(STEPS):
        sc_i, b = steps[k]
        pltpu.make_async_copy(
            x_hbm.at[x_rows(k)], bx[k % NBUF], si[k % NBUF]).wait()
        if b == 0:
            pltpu.make_async_copy(
                pos_hbm.at[pos_rows(sc_i)], bp0, sp0).wait()

        xb = bx[k % NBUF]

        @plsc.parallel_loop(0, R * D, step=16, unroll=8)
        def _add(i):
            r = i >> 10
            c = pl.multiple_of(i & (D - 1), 16)
            plsc.addupdate(xb.at[r, pl.ds(c, 16)], bp0[r, pl.ds(c, 16)])

        pltpu.async_copy(xb, out_hbm.at[x_rows(k)], so[k % NBUF])

        if b == B - 1 and sc_i + 1 < NSC:
            pltpu.async_copy(pos_hbm.at[pos_rows(sc_i + 1)], bp0, sp0)

        nk = k + PF
        if nk < STEPS:
            ko = nk - NBUF
            if ko >= 0:
                pltpu.make_async_copy(
                    bx[ko % NBUF], out_hbm.at[x_rows(ko)],
                    so[ko % NBUF]).wait()
            pltpu.async_copy(x_hbm.at[x_rows(nk)], bx[nk % NBUF], si[nk % NBUF])

    for k in range(STEPS - NBUF, STEPS):
        pltpu.make_async_copy(
            bx[k % NBUF], out_hbm.at[x_rows(k)], so[k % NBUF]).wait()


@jax.jit
def _sc_call(x2, pos_table):
    mesh = plsc.VectorSubcoreMesh(core_axis_name="c", subcore_axis_name="s")
    return pl.kernel(
        _sc_body,
        out_type=jax.ShapeDtypeStruct((B * S, D), jnp.float32),
        mesh=mesh,
        scratch_types=(
            [pltpu.VMEM((R, D), jnp.float32)] * (NBUF + 1)
            + [pltpu.SemaphoreType.DMA] * (2 * NBUF + 1)
        ),
        compiler_params=pltpu.CompilerParams(use_tc_tiling_on_sc=True),
    )(x2, pos_table)


def kernel(x, pos_table):
    out = _sc_call(x.reshape(B * S, D), pos_table)
    return out.reshape(B, S, D)


# speedup vs baseline: 1.4637x; 1.0761x over previous
import jax
import jax.numpy as jnp
from jax import lax
from jax.experimental import pallas as pl
from jax.experimental.pallas import tpu as pltpu
from jax.experimental.pallas import tpu_sc as plsc

B, S, D = 4, 4096, 1024
NC, NS = 2, 16
NW = NC * NS
SW = S // NW
R = 16
NSC = SW // R
STEPS = NSC * B
NBUF = 6
PF = 3


def _sc_body(x_hbm, pos_hbm, out_hbm,
             bx0, bx1, bx2, bx3, bx4, bx5, bp0, bp1,
             si0, si1, si2, si3, si4, si5,
             so0, so1, so2, so3, so4, so5, sp0, sp1):
    bx = (bx0, bx1, bx2, bx3, bx4, bx5)
    bp = (bp0, bp1)
    si = (si0, si1, si2, si3, si4, si5)
    so = (so0, so1, so2, so3, so4, so5)
    sp = (sp0, sp1)
    wid = lax.axis_index("s") * NC + lax.axis_index("c")
    s0 = wid * SW

    steps = [(sc_i, b) for sc_i in range(NSC) for b in range(B)]

    def x_rows(k):
        sc_i, b = steps[k]
        return pl.ds(b * S + s0 + sc_i * R, R)

    def pos_rows(sc_i):
        return pl.ds(s0 + sc_i * R, R)

    pltpu.async_copy(pos_hbm.at[pos_rows(0)], bp[0], sp[0])
    pltpu.async_copy(pos_hbm.at[pos_rows(1)], bp[1], sp[1])
    for k in range(PF):
        pltpu.async_copy(x_hbm.at[x_rows(k)], bx[k % NBUF], si[k % NBUF])

    for k in range(STEPS):
        sc_i, b = steps[k]
        pltpu.make_async_copy(
            x_hbm.at[x_rows(k)], bx[k % NBUF], si[k % NBUF]).wait()
        if b == 0:
            pltpu.make_async_copy(
                pos_hbm.at[pos_rows(sc_i)], bp[sc_i % 2], sp[sc_i % 2]).wait()

        xb = bx[k % NBUF]
        pb = bp[sc_i % 2]

        @plsc.parallel_loop(0, R * D, step=16, unroll=8)
        def _add(i):
            r = i >> 10
            c = pl.multiple_of(i & (D - 1), 16)
            plsc.addupdate(xb.at[r, pl.ds(c, 16)], pb[r, pl.ds(c, 16)])

        pltpu.async_copy(xb, out_hbm.at[x_rows(k)], so[k % NBUF])

        nk = k + PF
        if nk < STEPS:
            ko = nk - NBUF
            if ko >= 0:
                pltpu.make_async_copy(
                    bx[ko % NBUF], out_hbm.at[x_rows(ko)],
                    so[ko % NBUF]).wait()
            pltpu.async_copy(x_hbm.at[x_rows(nk)], bx[nk % NBUF], si[nk % NBUF])
            nsc, nb = steps[nk]
            if nb == 0 and nsc >= 2:
                pltpu.async_copy(
                    pos_hbm.at[pos_rows(nsc)], bp[nsc % 2], sp[nsc % 2])

    for k in range(STEPS - NBUF, STEPS):
        pltpu.make_async_copy(
            bx[k % NBUF], out_hbm.at[x_rows(k)], so[k % NBUF]).wait()


@jax.jit
def _sc_call(x2, pos_table):
    mesh = plsc.VectorSubcoreMesh(core_axis_name="c", subcore_axis_name="s")
    return pl.kernel(
        _sc_body,
        out_type=jax.ShapeDtypeStruct((B * S, D), jnp.float32),
        mesh=mesh,
        scratch_types=(
            [pltpu.VMEM((R, D), jnp.float32)] * (NBUF + 2)
            + [pltpu.SemaphoreType.DMA] * (2 * NBUF + 2)
        ),
        compiler_params=pltpu.CompilerParams(use_tc_tiling_on_sc=True),
    )(x2, pos_table)


def kernel(x, pos_table):
    out = _sc_call(x.reshape(B * S, D), pos_table)
    return out.reshape(B, S, D)
